# Initial kernel scaffold; baseline (speedup 1.0000x reference)
import jax
import jax.numpy as jnp
from jax.experimental import pallas as pl


def kernel(partial_charge, atomic_number, degree, ring_encoding, edge_index, W_in, b_in, W_att, a_src, a_dst, W1, b1, W_out, b_out):
    raise NotImplementedError("write your pallas kernel here")


# speedup vs baseline: 16.9533x; 16.9533x over previous
import jax
import jax.numpy as jnp
from jax import lax
from jax.experimental import pallas as pl
from jax.experimental.pallas import tpu as pltpu
from jax.experimental.pallas import tpu_sc as plsc

N = 100000
NP = 100096
TS = NP // 16
TSQ = TS // 2
E = 1600000
EP = 1638400
EW = 2048
NWIN = EP // (32 * EW)
ROWS_PER_TILE = EP // 32 // 128
H = 32
HQ = 8
F32 = jnp.float32
I32 = jnp.int32

_HIGH = jax.lax.Precision.HIGHEST


def _elu(v):
    return jnp.where(v > 0, v, jnp.exp(v) - 1.0)


def _tc1_body(feats, w_in, b_in, w_att, a_s, a_d,
              x_o, h0_o, h1_o, h2_o, h3_o, as_o, ad_o):
    x = jnp.dot(feats[...], w_in[...], precision=_HIGH) + b_in[...]
    x = _elu(x)
    h = jnp.dot(x, w_att[...], precision=_HIGH)
    x_o[...] = x
    h0_o[...] = h[:, 0:8]
    h1_o[...] = h[:, 8:16]
    h2_o[...] = h[:, 16:24]
    h3_o[...] = h[:, 24:32]
    as_o[...] = jnp.dot(h, a_s[...], precision=_HIGH)
    ad_o[...] = jnp.dot(h, a_d[...], precision=_HIGH)


def _tc1(feats, w_in, b_in, w_att, a_s, a_d):
    nb = N // 2000
    return pl.pallas_call(
        _tc1_body,
        grid=(nb,),
        in_specs=[
            pl.BlockSpec((2000, 48), lambda i: (i, 0)),
            pl.BlockSpec((48, H), lambda i: (0, 0)),
            pl.BlockSpec((1, H), lambda i: (0, 0)),
            pl.BlockSpec((H, H), lambda i: (0, 0)),
            pl.BlockSpec((H, 1), lambda i: (0, 0)),
            pl.BlockSpec((H, 1), lambda i: (0, 0)),
        ],
        out_specs=[
            pl.BlockSpec((2000, H), lambda i: (i, 0)),
            pl.BlockSpec((2000, HQ), lambda i: (i, 0)),
            pl.BlockSpec((2000, HQ), lambda i: (i, 0)),
            pl.BlockSpec((2000, HQ), lambda i: (i, 0)),
            pl.BlockSpec((2000, HQ), lambda i: (i, 0)),
            pl.BlockSpec((2000, 1), lambda i: (i, 0)),
            pl.BlockSpec((2000, 1), lambda i: (i, 0)),
        ],
        out_shape=[
            jax.ShapeDtypeStruct((N, H), F32),
            jax.ShapeDtypeStruct((N, HQ), F32),
            jax.ShapeDtypeStruct((N, HQ), F32),
            jax.ShapeDtypeStruct((N, HQ), F32),
            jax.ShapeDtypeStruct((N, HQ), F32),
            jax.ShapeDtypeStruct((N, 1), F32),
            jax.ShapeDtypeStruct((N, 1), F32),
        ],
    )(feats, w_in, b_in, w_att, a_s, a_d)


def _sc_body(srcR, dstR, als, ald, hq0, hq1, hq2, hq3,
             agg0, agg1, agg2, agg3, den_o, p_o,
             sidx, didx, abuf, bbuf, pbuf, rows, bounce, dbounce,
             as_sp, ad_sp, agg_sp, den_sp,
             sem0, sem1, sem2):
    c = lax.axis_index("c")
    s = lax.axis_index("s")
    wid = c * 16 + s
    nb = s * TS
    iota = jnp.arange(16, dtype=I32)
    rdiv = iota // 8
    rmod = iota % 8
    zeros16 = jnp.zeros((16,), F32)

    pltpu.sync_copy(als.at[pl.ds(nb, TS)], dbounce)
    pltpu.sync_copy(dbounce, as_sp.at[pl.ds(nb, TS)])
    pltpu.sync_copy(ald.at[pl.ds(nb, TS)], dbounce)
    pltpu.sync_copy(dbounce, ad_sp.at[pl.ds(nb, TS)])

    def _z1(i, _):
        dbounce[pl.ds(i * 16, 16)] = zeros16
        return 0
    lax.fori_loop(0, TS // 16, _z1, 0)
    pltpu.sync_copy(dbounce, den_sp.at[pl.ds(nb, TS)])

    def _zero_bounce():
        def _zb(i, _):
            plsc.store_scatter(bounce, [2 * i + rdiv, rmod], zeros16)
            return 0
        lax.fori_loop(0, TSQ // 2, _zb, 0)

    def _zero_agg():
        for k in range(2):
            pltpu.sync_copy(bounce, agg_sp.at[pl.ds(nb + k * TSQ, TSQ)])

    _zero_bounce()
    _zero_agg()

    hqs = (hq0, hq1, hq2, hq3)
    aggs = (agg0, agg1, agg2, agg3)
    for q in range(4):
        plsc.subcore_barrier()

        def _win(w, _, q=q):
            r0 = wid * ROWS_PER_TILE + w * 16
            base = wid * (NWIN * EW) + w * EW
            pltpu.sync_copy(srcR.at[pl.ds(r0, 16)], sidx)
            pltpu.sync_copy(dstR.at[pl.ds(r0, 16)], didx)
            if q == 0:
                def _g1(j, _):
                    pltpu.async_copy(as_sp.at[sidx.at[j]],
                                     abuf.at[pl.ds(j * 128, 128)], sem0)
                    pltpu.async_copy(ad_sp.at[didx.at[j]],
                                     bbuf.at[pl.ds(j * 128, 128)], sem0)
                    return 0
                lax.fori_loop(0, 16, _g1, 0)
                pltpu.make_async_copy(als.at[pl.ds(0, EW)], abuf, sem0).wait()
                pltpu.make_async_copy(als.at[pl.ds(0, EW)], bbuf, sem0).wait()

                def _pc(i, _):
                    t = abuf[pl.ds(i * 16, 16)] + bbuf[pl.ds(i * 16, 16)]
                    pbuf[pl.ds(i * 16, 16)] = jnp.exp(jnp.maximum(t, 0.2 * t))
                    return 0
                lax.fori_loop(0, EW // 16, _pc, 0)
                pltpu.sync_copy(pbuf, p_o.at[pl.ds(base, EW)])
            else:
                pltpu.sync_copy(p_o.at[pl.ds(base, EW)], pbuf)

            def _g2(j, _):
                pltpu.async_copy(hqs[q].at[sidx.at[j]],
                                 rows.at[pl.ds(j * 128, 128)], sem1)
                return 0
            lax.fori_loop(0, 16, _g2, 0)
            pltpu.make_async_copy(hqs[0].at[pl.ds(0, EW)], rows, sem1).wait()

            def _scale(i, _):
                ridx = 2 * i + rdiv
                pv = plsc.load_gather(pbuf, [ridx])
                hv = plsc.load_gather(rows, [ridx, rmod])
                plsc.store_scatter(rows, [ridx, rmod], hv * pv)
                return 0
            lax.fori_loop(0, EW // 2, _scale, 0)

            def _sc1(j, _):
                pltpu.async_copy(rows.at[pl.ds(j * 128, 128)],
                                 agg_sp.at[didx.at[j]], sem2, add=True)
                return 0
            lax.fori_loop(0, 16, _sc1, 0)
            if q == 0:
                def _sc2(j, _):
                    pltpu.async_copy(pbuf.at[pl.ds(j * 128, 128)],
                                     den_sp.at[didx.at[j]], sem2, add=True)
                    return 0
                lax.fori_loop(0, 16, _sc2, 0)
            pltpu.make_async_copy(hqs[0].at[pl.ds(0, EW)], rows, sem2).wait()
            if q == 0:
                pltpu.make_async_copy(als.at[pl.ds(0, EW)], pbuf, sem2).wait()
            return 0

        lax.fori_loop(0, NWIN, _win, 0)
        plsc.subcore_barrier()

        for k in range(2):
            pltpu.sync_copy(agg_sp.at[pl.ds(nb + k * TSQ, TSQ)], bounce)
            pltpu.sync_copy(bounce, aggs[q].at[c, pl.ds(nb + k * TSQ, TSQ)])
        if q < 3:
            _zero_bounce()
            _zero_agg()

    pltpu.sync_copy(den_sp.at[pl.ds(nb, TS)], dbounce)
    pltpu.sync_copy(dbounce, den_o.at[pl.ds(c * NP + nb, TS)])


def _sc_edge(srcR, dstR, als, ald, hq0, hq1, hq2, hq3):
    mesh = plsc.VectorSubcoreMesh(core_axis_name="c", subcore_axis_name="s")
    agg_t = jax.ShapeDtypeStruct((2, NP, HQ), F32)
    fn = pl.kernel(
        _sc_body,
        out_type=(
            agg_t, agg_t, agg_t, agg_t,
            jax.ShapeDtypeStruct((2 * NP,), F32),
            jax.ShapeDtypeStruct((EP,), F32),
        ),
        mesh=mesh,
        compiler_params=pltpu.CompilerParams(needs_layout_passes=False,
                                            use_tc_tiling_on_sc=False),
        scratch_types=(
            pltpu.VMEM((16, 128), I32),
            pltpu.VMEM((16, 128), I32),
            pltpu.VMEM((EW,), F32),
            pltpu.VMEM((EW,), F32),
            pltpu.VMEM((EW,), F32),
            pltpu.VMEM((EW, HQ), F32),
            pltpu.VMEM((TSQ, HQ), F32),
            pltpu.VMEM((TS,), F32),
            pltpu.VMEM_SHARED((NP,), F32),
            pltpu.VMEM_SHARED((NP,), F32),
            pltpu.VMEM_SHARED((NP, HQ), F32),
            pltpu.VMEM_SHARED((NP,), F32),
            pltpu.SemaphoreType.DMA,
            pltpu.SemaphoreType.DMA,
            pltpu.SemaphoreType.DMA,
        ),
    )
    return fn(srcR, dstR, als, ald, hq0, hq1, hq2, hq3)


def _tc2_body(x, a00, a01, a10, a11, a20, a21, a30, a31, d0, d1,
              w1, bb1, w_out, b_out, o_ref):
    den = d0[...] + d1[...] + 1e-16
    agg = jnp.concatenate(
        [a00[...] + a01[...], a10[...] + a11[...],
         a20[...] + a21[...], a30[...] + a31[...]], axis=1) / den
    hn = x[...] + agg
    sv = _elu(hn)
    sv = jnp.dot(sv, w1[...], precision=_HIGH) + bb1[...]
    sv = _elu(sv)
    o_ref[...] = jnp.dot(sv, w_out[...], precision=_HIGH) + b_out[...]


def _tc2(x, aggq, d0, d1, w1, b1v, w_out, b_out):
    nb = N // 2000
    bq = pl.BlockSpec((2000, HQ), lambda i: (i, 0))
    return pl.pallas_call(
        _tc2_body,
        grid=(nb,),
        in_specs=[
            pl.BlockSpec((2000, H), lambda i: (i, 0)),
            bq, bq, bq, bq, bq, bq, bq, bq,
            pl.BlockSpec((2000, 1), lambda i: (i, 0)),
            pl.BlockSpec((2000, 1), lambda i: (i, 0)),
            pl.BlockSpec((H, H), lambda i: (0, 0)),
            pl.BlockSpec((1, H), lambda i: (0, 0)),
            pl.BlockSpec((H, 6), lambda i: (0, 0)),
            pl.BlockSpec((1, 6), lambda i: (0, 0)),
        ],
        out_specs=pl.BlockSpec((2000, 6), lambda i: (i, 0)),
        out_shape=jax.ShapeDtypeStruct((N, 6), F32),
    )(x, *aggq, d0, d1, w1, b1v, w_out, b_out)


def kernel(partial_charge, atomic_number, degree, ring_encoding, edge_index,
           W_in, b_in, W_att, a_src, a_dst, W1, b1, W_out, b_out):
    src = edge_index[0].astype(I32)
    dst = edge_index[1].astype(I32)
    npad = EP - E
    src_p = jnp.concatenate([src, jnp.zeros((npad,), I32)]).reshape(EP // 128, 128)
    dst_p = jnp.concatenate(
        [dst, N + (jnp.arange(npad, dtype=I32) % 16)]).reshape(EP // 128, 128)

    feats = jnp.concatenate(
        [partial_charge, atomic_number, degree, ring_encoding], axis=1)
    feats = jnp.pad(feats, ((0, 0), (0, 48 - feats.shape[1])))
    w_in_p = jnp.pad(W_in, ((0, 48 - W_in.shape[0]), (0, 0)))

    x, h0, h1, h2, h3, als, ald = _tc1(
        feats, w_in_p, b_in.reshape(1, H), W_att,
        a_src.reshape(H, 1), a_dst.reshape(H, 1))
    als_p = jnp.pad(als.reshape(-1), (0, NP - N))
    ald_p = jnp.pad(ald.reshape(-1), (0, NP - N))

    a0, a1, a2, a3, den, _ = _sc_edge(src_p, dst_p, als_p, ald_p, h0, h1, h2, h3)

    aggq = [a0[0, :N], a0[1, :N], a1[0, :N], a1[1, :N],
            a2[0, :N], a2[1, :N], a3[0, :N], a3[1, :N]]
    return _tc2(x, aggq, den[:N, None], den[NP:NP + N, None],
                W1, b1.reshape(1, H), W_out, b_out.reshape(1, 6))
